# baseline (speedup 1.0000x reference)
import functools

import jax
import jax.numpy as jnp
from jax import lax
from jax.experimental import pallas as pl
from jax.experimental.pallas import tpu as pltpu
from jax.experimental.pallas import tpu_sc as plsc

NC = 2
NS = 16
LANES = 16

H = 8
C = 32
KA = 40
KB = 80


def _bcast_lane(v, lane):
    return lax.gather(
        v, jnp.full((LANES, 1), lane, jnp.int32),
        lax.GatherDimensionNumbers(offset_dims=(),
                                   collapsed_slice_dims=(0,),
                                   start_index_map=(0,)),
        slice_sizes=(1,),
        mode=lax.GatherScatterMode.PROMISE_IN_BOUNDS)


def _zero_rows(ref, nrows, width):
    @pl.loop(0, nrows)
    def _z(i):
        z = jnp.zeros((LANES,), jnp.float32)
        for j in range(width // LANES):
            ref[i, pl.ds(j * LANES, LANES)] = z


def _stage1_body(x_ref, w_ref, as_ref, ad_ref,
                 hlo_ref, hhi_ref, asrc_ref, adst_ref):
    h = jnp.dot(x_ref[...], w_ref[...], preferred_element_type=jnp.float32)
    hlo_ref[...] = h[:, :128]
    hhi_ref[...] = h[:, 128:]
    asrc_ref[...] = jnp.dot(h, as_ref[...], preferred_element_type=jnp.float32)
    adst_ref[...] = jnp.dot(h, ad_ref[...], preferred_element_type=jnp.float32)


def _stage1(x, W, As128, Ad128, bn):
    n, d_in = x.shape
    d_out = W.shape[1]
    grid = (n // bn,)
    return pl.pallas_call(
        _stage1_body,
        grid=grid,
        in_specs=[
            pl.BlockSpec((bn, d_in), lambda i: (i, 0)),
            pl.BlockSpec((d_in, d_out), lambda i: (0, 0)),
            pl.BlockSpec((d_out, 128), lambda i: (0, 0)),
            pl.BlockSpec((d_out, 128), lambda i: (0, 0)),
        ],
        out_specs=[
            pl.BlockSpec((bn, 128), lambda i: (i, 0)),
            pl.BlockSpec((bn, 128), lambda i: (i, 0)),
            pl.BlockSpec((bn, 128), lambda i: (i, 0)),
            pl.BlockSpec((bn, 128), lambda i: (i, 0)),
        ],
        out_shape=[
            jax.ShapeDtypeStruct((n, 128), jnp.float32),
            jax.ShapeDtypeStruct((n, 128), jnp.float32),
            jax.ShapeDtypeStruct((n, 128), jnp.float32),
            jax.ShapeDtypeStruct((n, 128), jnp.float32),
        ],
    )(x, W, As128, Ad128)


def _logit_body(n, e, ch, rows_main,
                asrc, adst, src, dst,
                eal, dnm0, dnm1,
                sidx0, sidx1, didx0, didx1, sa0, sa1, da0, da1,
                ebuf0, ebuf1, pbuf, dnm,
                isem0, isem1, gsem0, gsem1, wsem0, wsem1):
    c = lax.axis_index("c")
    s = lax.axis_index("s")
    w = c * NS + s
    rows_last = n - rows_main * (NS - 1)
    sidx = (sidx0, sidx1)
    didx = (didx0, didx1)
    sa = (sa0, sa1)
    da = (da0, da1)
    ebuf = (ebuf0, ebuf1)
    isem = (isem0, isem1)
    gsem = (gsem0, gsem1)
    wsem = (wsem0, wsem1)

    _zero_rows(pbuf, KA, 128)
    row0 = s * rows_main

    def _zero_spmem(nrows):
        off = 0
        while off < nrows:
            step = min(KA, nrows - off)
            pltpu.sync_copy(pbuf.at[pl.ds(0, step)],
                            dnm.at[pl.ds(row0 + off, step)])
            off += step

    @pl.when(s < NS - 1)
    def _():
        _zero_spmem(rows_main)

    @pl.when(s == NS - 1)
    def _():
        _zero_spmem(rows_last)

    plsc.subcore_barrier()

    ep = e // (NC * NS)

    def _idx_copies(i, p):
        base = w * ep + i * KA
        return (pltpu.make_async_copy(src.at[pl.ds(base, KA)], sidx[p],
                                      isem[p]),
                pltpu.make_async_copy(dst.at[pl.ds(base, KA)], didx[p],
                                      isem[p]))

    def _issue_idx(i, p):
        for d in _idx_copies(i, p):
            d.start()

    def _wait_idx(i, p):
        for d in _idx_copies(i, p):
            d.wait()

    def _gather_copies(p):
        return (pltpu.make_async_copy(asrc.at[sidx[p]], sa[p], gsem[p]),
                pltpu.make_async_copy(adst.at[didx[p]], da[p], gsem[p]))

    def _issue_gather(p):
        for d in _gather_copies(p):
            d.start()

    def _wait_gather(p):
        for d in _gather_copies(p):
            d.wait()

    def _eal_write(i, p):
        base = w * ep + i * KA
        return pltpu.make_async_copy(ebuf[p], eal.at[pl.ds(base, KA)],
                                     wsem[p])

    def _body(i, p):
        @pl.when(i + 1 < ch)
        def _():
            _wait_idx(i + 1, 1 - p)
            _issue_gather(1 - p)

        _wait_gather(p)
        @pl.when(i >= 2)
        def _():
            _eal_write(i - 2, p).wait()

        @pl.loop(0, KA, unroll=4)
        def _edge(j):
            al = sa[p][j, pl.ds(0, LANES)] + da[p][j, pl.ds(0, LANES)]
            ea = jnp.exp(jnp.maximum(al, 0.2 * al))
            ebuf[p][j, :] = ea
            pbuf[j, pl.ds(0, LANES)] = ea

        _eal_write(i, p).start()
        pltpu.sync_copy(pbuf, dnm.at[didx[p]], add=True)

        @pl.when(i + 2 < ch)
        def _():
            _issue_idx(i + 2, p)

    _issue_idx(0, 0)
    _issue_idx(1, 1)
    _wait_idx(0, 0)
    _issue_gather(0)

    @pl.loop(0, ch)
    def _chunk(i):
        @pl.when(i % 2 == 0)
        def _():
            _body(i, 0)

        @pl.when(i % 2 == 1)
        def _():
            _body(i, 1)

    for last in (ch - 2, ch - 1):
        if last >= 0:
            _eal_write(last, last % 2).wait()

    plsc.subcore_barrier()

    def _writeout(nrows):
        @pl.when(c == 0)
        def _():
            pltpu.sync_copy(dnm.at[pl.ds(row0, nrows)],
                            dnm0.at[pl.ds(row0, nrows)])

        @pl.when(c == 1)
        def _():
            pltpu.sync_copy(dnm.at[pl.ds(row0, nrows)],
                            dnm1.at[pl.ds(row0, nrows)])

    @pl.when(s < NS - 1)
    def _():
        _writeout(rows_main)

    @pl.when(s == NS - 1)
    def _():
        _writeout(rows_last)


def _stage2a(asrc_p, adst_p, src, dst):
    n = asrc_p.shape[0]
    e = src.shape[0]
    ep = e // (NC * NS)
    ch = ep // KA
    rows_main = ((n // NS) // 8) * 8
    mesh = plsc.VectorSubcoreMesh(core_axis_name="c", subcore_axis_name="s",
                                  num_cores=NC, num_subcores=NS)
    body = functools.partial(_logit_body, n, e, ch, rows_main)
    return pl.kernel(
        body,
        out_type=[
            jax.ShapeDtypeStruct((e, 16), jnp.float32),
            jax.ShapeDtypeStruct((n, 128), jnp.float32),
            jax.ShapeDtypeStruct((n, 128), jnp.float32),
        ],
        mesh=mesh,
        scratch_types=[
            pltpu.VMEM((KA,), jnp.int32),
            pltpu.VMEM((KA,), jnp.int32),
            pltpu.VMEM((KA,), jnp.int32),
            pltpu.VMEM((KA,), jnp.int32),
            pltpu.VMEM((KA, 128), jnp.float32),
            pltpu.VMEM((KA, 128), jnp.float32),
            pltpu.VMEM((KA, 128), jnp.float32),
            pltpu.VMEM((KA, 128), jnp.float32),
            pltpu.VMEM((KA, 16), jnp.float32),
            pltpu.VMEM((KA, 16), jnp.float32),
            pltpu.VMEM((KA, 128), jnp.float32),
            pltpu.VMEM_SHARED((n, 128), jnp.float32),
            pltpu.SemaphoreType.DMA,
            pltpu.SemaphoreType.DMA,
            pltpu.SemaphoreType.DMA,
            pltpu.SemaphoreType.DMA,
            pltpu.SemaphoreType.DMA,
            pltpu.SemaphoreType.DMA,
        ],
    )(asrc_p, adst_p, src, dst)


def _msg_body(n, e, ch, rows_main,
              hlo, hhi, eal, src, dst,
              out0, out1,
              sidx0, sidx1, didx0, didx1, hrow0, hrow1, ebuf0, ebuf1, acc,
              isem0, isem1, gsem0, gsem1):
    c = lax.axis_index("c")
    s = lax.axis_index("s")
    rows_last = n - rows_main * (NS - 1)
    sidx = (sidx0, sidx1)
    didx = (didx0, didx1)
    hrow = (hrow0, hrow1)
    ebuf = (ebuf0, ebuf1)
    isem = (isem0, isem1)
    gsem = (gsem0, gsem1)

    _zero_rows(hrow0, KB, 128)
    row0 = s * rows_main

    def _zero_spmem(nrows):
        off = 0
        while off < nrows:
            step = min(KB, nrows - off)
            pltpu.sync_copy(hrow0.at[pl.ds(0, step)],
                            acc.at[pl.ds(row0 + off, step)])
            off += step

    @pl.when(s < NS - 1)
    def _():
        _zero_spmem(rows_main)

    @pl.when(s == NS - 1)
    def _():
        _zero_spmem(rows_last)

    plsc.subcore_barrier()

    ep = e // NS

    def _idx_copies(i, p):
        base = s * ep + i * KB
        return (pltpu.make_async_copy(src.at[pl.ds(base, KB)], sidx[p],
                                      isem[p]),
                pltpu.make_async_copy(dst.at[pl.ds(base, KB)], didx[p],
                                      isem[p]),
                pltpu.make_async_copy(eal.at[pl.ds(base, KB)], ebuf[p],
                                      isem[p]))

    def _issue_idx(i, p):
        for d in _idx_copies(i, p):
            d.start()

    def _wait_idx(i, p):
        for d in _idx_copies(i, p):
            d.wait()

    def _gather_copies(p):
        return (pltpu.make_async_copy(hlo.at[sidx[p]], hrow[p], gsem[p]),
                pltpu.make_async_copy(hhi.at[sidx[p]], hrow[p], gsem[p]))

    def _issue_gather(p):
        lo, hi = _gather_copies(p)

        @pl.when(c == 0)
        def _():
            lo.start()

        @pl.when(c == 1)
        def _():
            hi.start()

    def _wait_gather(p):
        _gather_copies(p)[0].wait()

    def _body(i, p):
        @pl.when(i + 1 < ch)
        def _():
            _wait_idx(i + 1, 1 - p)
            _issue_gather(1 - p)

        _wait_gather(p)

        @pl.loop(0, KB, unroll=4)
        def _edge(j):
            ea = ebuf[p][j, :]
            for hd in range(4):
                wv = _bcast_lane(ea, 4 * c + hd)
                col = hd * C
                hrow[p][j, pl.ds(col, LANES)] = (
                    hrow[p][j, pl.ds(col, LANES)] * wv)
                hrow[p][j, pl.ds(col + LANES, LANES)] = (
                    hrow[p][j, pl.ds(col + LANES, LANES)] * wv)

        pltpu.sync_copy(hrow[p], acc.at[didx[p]], add=True)

        @pl.when(i + 2 < ch)
        def _():
            _issue_idx(i + 2, p)

    _issue_idx(0, 0)
    _issue_idx(1, 1)
    _wait_idx(0, 0)
    _issue_gather(0)

    @pl.loop(0, ch)
    def _chunk(i):
        @pl.when(i % 2 == 0)
        def _():
            _body(i, 0)

        @pl.when(i % 2 == 1)
        def _():
            _body(i, 1)

    plsc.subcore_barrier()

    def _writeout(nrows):
        @pl.when(c == 0)
        def _():
            pltpu.sync_copy(acc.at[pl.ds(row0, nrows)],
                            out0.at[pl.ds(row0, nrows)])

        @pl.when(c == 1)
        def _():
            pltpu.sync_copy(acc.at[pl.ds(row0, nrows)],
                            out1.at[pl.ds(row0, nrows)])

    @pl.when(s < NS - 1)
    def _():
        _writeout(rows_main)

    @pl.when(s == NS - 1)
    def _():
        _writeout(rows_last)


def _stage2b(hlo, hhi, eal, src, dst):
    n = hlo.shape[0]
    e = src.shape[0]
    ep = e // NS
    ch = ep // KB
    rows_main = ((n // NS) // 8) * 8
    mesh = plsc.VectorSubcoreMesh(core_axis_name="c", subcore_axis_name="s",
                                  num_cores=NC, num_subcores=NS)
    body = functools.partial(_msg_body, n, e, ch, rows_main)
    return pl.kernel(
        body,
        out_type=[
            jax.ShapeDtypeStruct((n, 128), jnp.float32),
            jax.ShapeDtypeStruct((n, 128), jnp.float32),
        ],
        mesh=mesh,
        scratch_types=[
            pltpu.VMEM((KB,), jnp.int32),
            pltpu.VMEM((KB,), jnp.int32),
            pltpu.VMEM((KB,), jnp.int32),
            pltpu.VMEM((KB,), jnp.int32),
            pltpu.VMEM((KB, 128), jnp.float32),
            pltpu.VMEM((KB, 128), jnp.float32),
            pltpu.VMEM((KB, 16), jnp.float32),
            pltpu.VMEM((KB, 16), jnp.float32),
            pltpu.VMEM_SHARED((n, 128), jnp.float32),
            pltpu.SemaphoreType.DMA,
            pltpu.SemaphoreType.DMA,
            pltpu.SemaphoreType.DMA,
            pltpu.SemaphoreType.DMA,
        ],
    )(hlo, hhi, eal, src, dst)


def _stage3_body(x_ref, hlo_ref, hhi_ref, asrc_ref, adst_ref, sc0_ref, sc1_ref,
                 dnm0_ref, dnm1_ref, exp_ref, bias_ref, lnw_ref, lnb_ref,
                 out_ref):
    al = asrc_ref[:, :16] + adst_ref[:, :16]
    el = jnp.exp(jnp.maximum(al, 0.2 * al))
    exp_mat = exp_ref[...]
    el_exp = jnp.dot(el, exp_mat, preferred_element_type=jnp.float32)
    dn = dnm0_ref[:, :16] + dnm1_ref[:, :16] + el
    dn_exp = jnp.dot(dn, exp_mat, preferred_element_type=jnp.float32)
    h = jnp.concatenate([hlo_ref[...], hhi_ref[...]], axis=-1)
    acc = jnp.concatenate([sc0_ref[...], sc1_ref[...]], axis=-1)
    out = (acc + el_exp * h) / (dn_exp + 1e-16)
    out = out + bias_ref[...] + x_ref[...]
    mu = jnp.mean(out, axis=-1, keepdims=True)
    var = jnp.mean((out - mu) ** 2, axis=-1, keepdims=True)
    out = (out - mu) * jax.lax.rsqrt(var + 1e-5)
    out_ref[...] = out * lnw_ref[...] + lnb_ref[...]


def _stage3(x, hlo, hhi, asrc_p, adst_p, sc0, sc1, dnm0, dnm1, exp_mat,
            bias, ln_w, ln_b, bn):
    n, d = x.shape
    grid = (n // bn,)
    vec = lambda i: (0,)
    return pl.pallas_call(
        _stage3_body,
        grid=grid,
        in_specs=[
            pl.BlockSpec((bn, d), lambda i: (i, 0)),
            pl.BlockSpec((bn, 128), lambda i: (i, 0)),
            pl.BlockSpec((bn, 128), lambda i: (i, 0)),
            pl.BlockSpec((bn, 128), lambda i: (i, 0)),
            pl.BlockSpec((bn, 128), lambda i: (i, 0)),
            pl.BlockSpec((bn, 128), lambda i: (i, 0)),
            pl.BlockSpec((bn, 128), lambda i: (i, 0)),
            pl.BlockSpec((bn, 128), lambda i: (i, 0)),
            pl.BlockSpec((bn, 128), lambda i: (i, 0)),
            pl.BlockSpec((16, d), lambda i: (0, 0)),
            pl.BlockSpec((d,), vec),
            pl.BlockSpec((d,), vec),
            pl.BlockSpec((d,), vec),
        ],
        out_specs=pl.BlockSpec((bn, d), lambda i: (i, 0)),
        out_shape=jax.ShapeDtypeStruct((n, d), jnp.float32),
    )(x, hlo, hhi, asrc_p, adst_p, sc0, sc1, dnm0, dnm1, exp_mat,
      bias, ln_w, ln_b)


def kernel(x, edge_index, W, att_src, att_dst, bias, ln_w, ln_b):
    n, d_in = x.shape
    d_out = W.shape[1]

    eye_h = jnp.eye(H, dtype=jnp.float32)
    As = (att_src[:, :, None] * eye_h[:, None, :]).reshape(d_out, H)
    Ad = (att_dst[:, :, None] * eye_h[:, None, :]).reshape(d_out, H)
    pad = jnp.zeros((d_out, 128 - H), jnp.float32)
    As128 = jnp.concatenate([As, pad], axis=1)
    Ad128 = jnp.concatenate([Ad, pad], axis=1)
    head_of_col = (jnp.arange(d_out, dtype=jnp.int32) // C)[None, :]
    exp_mat = (jnp.arange(16, dtype=jnp.int32)[:, None] == head_of_col
               ).astype(jnp.float32)

    src = edge_index[0]
    dst = edge_index[1]

    bn = 1000
    hlo, hhi, asrc_p, adst_p = _stage1(x, W, As128, Ad128, bn)
    eal, dnm0, dnm1 = _stage2a(asrc_p, adst_p, src, dst)
    sc0, sc1 = _stage2b(hlo, hhi, eal, src, dst)
    return _stage3(x, hlo, hhi, asrc_p, adst_p, sc0, sc1, dnm0, dnm1, exp_mat,
                   bias, ln_w, ln_b, bn)


# speedup vs baseline: 1.1689x; 1.1689x over previous
import functools

import jax
import jax.numpy as jnp
from jax import lax
from jax.experimental import pallas as pl
from jax.experimental.pallas import tpu as pltpu
from jax.experimental.pallas import tpu_sc as plsc

NC = 2
NS = 16
LANES = 16

H = 8
C = 32
KA = 40
KB = 80


def _bcast_lane(v, lane):
    return lax.gather(
        v, jnp.full((LANES, 1), lane, jnp.int32),
        lax.GatherDimensionNumbers(offset_dims=(),
                                   collapsed_slice_dims=(0,),
                                   start_index_map=(0,)),
        slice_sizes=(1,),
        mode=lax.GatherScatterMode.PROMISE_IN_BOUNDS)


def _zero_rows(ref, nrows, width):
    @pl.loop(0, nrows)
    def _z(i):
        z = jnp.zeros((LANES,), jnp.float32)
        for j in range(width // LANES):
            ref[i, pl.ds(j * LANES, LANES)] = z


def _stage1_body(x_ref, w_ref, as_ref, ad_ref,
                 hlo_ref, hhi_ref, asrc_ref, adst_ref):
    h = jnp.dot(x_ref[...], w_ref[...], preferred_element_type=jnp.float32)
    hlo_ref[...] = h[:, :128]
    hhi_ref[...] = h[:, 128:]
    asrc_ref[...] = jnp.dot(h, as_ref[...], preferred_element_type=jnp.float32)
    adst_ref[...] = jnp.dot(h, ad_ref[...], preferred_element_type=jnp.float32)


def _stage1(x, W, As128, Ad128, bn):
    n, d_in = x.shape
    d_out = W.shape[1]
    grid = (n // bn,)
    return pl.pallas_call(
        _stage1_body,
        grid=grid,
        in_specs=[
            pl.BlockSpec((bn, d_in), lambda i: (i, 0)),
            pl.BlockSpec((d_in, d_out), lambda i: (0, 0)),
            pl.BlockSpec((d_out, 128), lambda i: (0, 0)),
            pl.BlockSpec((d_out, 128), lambda i: (0, 0)),
        ],
        out_specs=[
            pl.BlockSpec((bn, 128), lambda i: (i, 0)),
            pl.BlockSpec((bn, 128), lambda i: (i, 0)),
            pl.BlockSpec((bn, 128), lambda i: (i, 0)),
            pl.BlockSpec((bn, 128), lambda i: (i, 0)),
        ],
        out_shape=[
            jax.ShapeDtypeStruct((n, 128), jnp.float32),
            jax.ShapeDtypeStruct((n, 128), jnp.float32),
            jax.ShapeDtypeStruct((n, 128), jnp.float32),
            jax.ShapeDtypeStruct((n, 128), jnp.float32),
        ],
    )(x, W, As128, Ad128)


def _logit_body(n, e, ch, rows_main,
                asrc, adst, src, dst,
                eal, dnm0, dnm1,
                sidx0, sidx1, didx0, didx1, sa0, sa1, da0, da1,
                ebuf0, ebuf1, pbuf, dnm,
                isem0, isem1, gsem0, gsem1, wsem0, wsem1):
    c = lax.axis_index("c")
    s = lax.axis_index("s")
    w = c * NS + s
    rows_last = n - rows_main * (NS - 1)
    sidx = (sidx0, sidx1)
    didx = (didx0, didx1)
    sa = (sa0, sa1)
    da = (da0, da1)
    ebuf = (ebuf0, ebuf1)
    isem = (isem0, isem1)
    gsem = (gsem0, gsem1)
    wsem = (wsem0, wsem1)

    _zero_rows(pbuf, KA, 128)
    row0 = s * rows_main

    def _zero_spmem(nrows):
        off = 0
        while off < nrows:
            step = min(KA, nrows - off)
            pltpu.sync_copy(pbuf.at[pl.ds(0, step)],
                            dnm.at[pl.ds(row0 + off, step)])
            off += step

    @pl.when(s < NS - 1)
    def _():
        _zero_spmem(rows_main)

    @pl.when(s == NS - 1)
    def _():
        _zero_spmem(rows_last)

    plsc.subcore_barrier()

    ep = e // (NC * NS)

    def _idx_copies(i, p):
        base = w * ep + i * KA
        return (pltpu.make_async_copy(src.at[pl.ds(base, KA)], sidx[p],
                                      isem[p]),
                pltpu.make_async_copy(dst.at[pl.ds(base, KA)], didx[p],
                                      isem[p]))

    def _issue_idx(i, p):
        for d in _idx_copies(i, p):
            d.start()

    def _wait_idx(i, p):
        for d in _idx_copies(i, p):
            d.wait()

    def _gather_copies(p):
        return (pltpu.make_async_copy(asrc.at[sidx[p]], sa[p], gsem[p]),
                pltpu.make_async_copy(adst.at[didx[p]], da[p], gsem[p]))

    def _issue_gather(p):
        for d in _gather_copies(p):
            d.start()

    def _wait_gather(p):
        for d in _gather_copies(p):
            d.wait()

    def _eal_write(i, p):
        base = w * ep + i * KA
        return pltpu.make_async_copy(ebuf[p], eal.at[pl.ds(base, KA)],
                                     wsem[p])

    def _body(i, p):
        @pl.when(i + 1 < ch)
        def _():
            _wait_idx(i + 1, 1 - p)
            _issue_gather(1 - p)

        _wait_gather(p)
        @pl.when(i >= 2)
        def _():
            _eal_write(i - 2, p).wait()

        @plsc.parallel_loop(0, KA, unroll=2)
        def _edge(j):
            al = sa[p][j, pl.ds(0, LANES)] + da[p][j, pl.ds(0, LANES)]
            ea = jnp.exp(jnp.maximum(al, 0.2 * al))
            ebuf[p][j, :] = ea
            pbuf[j, pl.ds(0, LANES)] = ea

        _eal_write(i, p).start()
        pltpu.sync_copy(pbuf, dnm.at[didx[p]], add=True)

        @pl.when(i + 2 < ch)
        def _():
            _issue_idx(i + 2, p)

    _issue_idx(0, 0)
    _issue_idx(1, 1)
    _wait_idx(0, 0)
    _issue_gather(0)

    @pl.loop(0, ch)
    def _chunk(i):
        @pl.when(i % 2 == 0)
        def _():
            _body(i, 0)

        @pl.when(i % 2 == 1)
        def _():
            _body(i, 1)

    for last in (ch - 2, ch - 1):
        if last >= 0:
            _eal_write(last, last % 2).wait()

    plsc.subcore_barrier()

    def _writeout(nrows):
        @pl.when(c == 0)
        def _():
            pltpu.sync_copy(dnm.at[pl.ds(row0, nrows)],
                            dnm0.at[pl.ds(row0, nrows)])

        @pl.when(c == 1)
        def _():
            pltpu.sync_copy(dnm.at[pl.ds(row0, nrows)],
                            dnm1.at[pl.ds(row0, nrows)])

    @pl.when(s < NS - 1)
    def _():
        _writeout(rows_main)

    @pl.when(s == NS - 1)
    def _():
        _writeout(rows_last)


def _stage2a(asrc_p, adst_p, src, dst):
    n = asrc_p.shape[0]
    e = src.shape[0]
    ep = e // (NC * NS)
    ch = ep // KA
    rows_main = ((n // NS) // 8) * 8
    mesh = plsc.VectorSubcoreMesh(core_axis_name="c", subcore_axis_name="s",
                                  num_cores=NC, num_subcores=NS)
    body = functools.partial(_logit_body, n, e, ch, rows_main)
    return pl.kernel(
        body,
        out_type=[
            jax.ShapeDtypeStruct((e, 16), jnp.float32),
            jax.ShapeDtypeStruct((n, 128), jnp.float32),
            jax.ShapeDtypeStruct((n, 128), jnp.float32),
        ],
        mesh=mesh,
        scratch_types=[
            pltpu.VMEM((KA,), jnp.int32),
            pltpu.VMEM((KA,), jnp.int32),
            pltpu.VMEM((KA,), jnp.int32),
            pltpu.VMEM((KA,), jnp.int32),
            pltpu.VMEM((KA, 128), jnp.float32),
            pltpu.VMEM((KA, 128), jnp.float32),
            pltpu.VMEM((KA, 128), jnp.float32),
            pltpu.VMEM((KA, 128), jnp.float32),
            pltpu.VMEM((KA, 16), jnp.float32),
            pltpu.VMEM((KA, 16), jnp.float32),
            pltpu.VMEM((KA, 128), jnp.float32),
            pltpu.VMEM_SHARED((n, 128), jnp.float32),
            pltpu.SemaphoreType.DMA,
            pltpu.SemaphoreType.DMA,
            pltpu.SemaphoreType.DMA,
            pltpu.SemaphoreType.DMA,
            pltpu.SemaphoreType.DMA,
            pltpu.SemaphoreType.DMA,
        ],
    )(asrc_p, adst_p, src, dst)


def _msg_body(n, e, ch, rows_main,
              hlo, hhi, eal, src, dst,
              out0, out1,
              sidx0, sidx1, didx0, didx1, hrow0, hrow1, ebuf0, ebuf1, acc,
              isem0, isem1, gsem0, gsem1):
    c = lax.axis_index("c")
    s = lax.axis_index("s")
    rows_last = n - rows_main * (NS - 1)
    sidx = (sidx0, sidx1)
    didx = (didx0, didx1)
    hrow = (hrow0, hrow1)
    ebuf = (ebuf0, ebuf1)
    isem = (isem0, isem1)
    gsem = (gsem0, gsem1)

    _zero_rows(hrow0, KB, 128)
    row0 = s * rows_main

    def _zero_spmem(nrows):
        off = 0
        while off < nrows:
            step = min(KB, nrows - off)
            pltpu.sync_copy(hrow0.at[pl.ds(0, step)],
                            acc.at[pl.ds(row0 + off, step)])
            off += step

    @pl.when(s < NS - 1)
    def _():
        _zero_spmem(rows_main)

    @pl.when(s == NS - 1)
    def _():
        _zero_spmem(rows_last)

    plsc.subcore_barrier()

    ep = e // NS

    def _idx_copies(i, p):
        base = s * ep + i * KB
        return (pltpu.make_async_copy(src.at[pl.ds(base, KB)], sidx[p],
                                      isem[p]),
                pltpu.make_async_copy(dst.at[pl.ds(base, KB)], didx[p],
                                      isem[p]),
                pltpu.make_async_copy(eal.at[pl.ds(base, KB)], ebuf[p],
                                      isem[p]))

    def _issue_idx(i, p):
        for d in _idx_copies(i, p):
            d.start()

    def _wait_idx(i, p):
        for d in _idx_copies(i, p):
            d.wait()

    def _gather_copies(p):
        return (pltpu.make_async_copy(hlo.at[sidx[p]], hrow[p], gsem[p]),
                pltpu.make_async_copy(hhi.at[sidx[p]], hrow[p], gsem[p]))

    def _issue_gather(p):
        lo, hi = _gather_copies(p)

        @pl.when(c == 0)
        def _():
            lo.start()

        @pl.when(c == 1)
        def _():
            hi.start()

    def _wait_gather(p):
        _gather_copies(p)[0].wait()

    def _body(i, p):
        @pl.when(i + 1 < ch)
        def _():
            _wait_idx(i + 1, 1 - p)
            _issue_gather(1 - p)

        _wait_gather(p)

        @plsc.parallel_loop(0, KB, unroll=2)
        def _edge(j):
            ea = ebuf[p][j, :]
            for hd in range(4):
                wv = _bcast_lane(ea, 4 * c + hd)
                col = hd * C
                hrow[p][j, pl.ds(col, LANES)] = (
                    hrow[p][j, pl.ds(col, LANES)] * wv)
                hrow[p][j, pl.ds(col + LANES, LANES)] = (
                    hrow[p][j, pl.ds(col + LANES, LANES)] * wv)

        pltpu.sync_copy(hrow[p], acc.at[didx[p]], add=True)

        @pl.when(i + 2 < ch)
        def _():
            _issue_idx(i + 2, p)

    _issue_idx(0, 0)
    _issue_idx(1, 1)
    _wait_idx(0, 0)
    _issue_gather(0)

    @pl.loop(0, ch)
    def _chunk(i):
        @pl.when(i % 2 == 0)
        def _():
            _body(i, 0)

        @pl.when(i % 2 == 1)
        def _():
            _body(i, 1)

    plsc.subcore_barrier()

    def _writeout(nrows):
        @pl.when(c == 0)
        def _():
            pltpu.sync_copy(acc.at[pl.ds(row0, nrows)],
                            out0.at[pl.ds(row0, nrows)])

        @pl.when(c == 1)
        def _():
            pltpu.sync_copy(acc.at[pl.ds(row0, nrows)],
                            out1.at[pl.ds(row0, nrows)])

    @pl.when(s < NS - 1)
    def _():
        _writeout(rows_main)

    @pl.when(s == NS - 1)
    def _():
        _writeout(rows_last)


def _stage2b(hlo, hhi, eal, src, dst):
    n = hlo.shape[0]
    e = src.shape[0]
    ep = e // NS
    ch = ep // KB
    rows_main = ((n // NS) // 8) * 8
    mesh = plsc.VectorSubcoreMesh(core_axis_name="c", subcore_axis_name="s",
                                  num_cores=NC, num_subcores=NS)
    body = functools.partial(_msg_body, n, e, ch, rows_main)
    return pl.kernel(
        body,
        out_type=[
            jax.ShapeDtypeStruct((n, 128), jnp.float32),
            jax.ShapeDtypeStruct((n, 128), jnp.float32),
        ],
        mesh=mesh,
        scratch_types=[
            pltpu.VMEM((KB,), jnp.int32),
            pltpu.VMEM((KB,), jnp.int32),
            pltpu.VMEM((KB,), jnp.int32),
            pltpu.VMEM((KB,), jnp.int32),
            pltpu.VMEM((KB, 128), jnp.float32),
            pltpu.VMEM((KB, 128), jnp.float32),
            pltpu.VMEM((KB, 16), jnp.float32),
            pltpu.VMEM((KB, 16), jnp.float32),
            pltpu.VMEM_SHARED((n, 128), jnp.float32),
            pltpu.SemaphoreType.DMA,
            pltpu.SemaphoreType.DMA,
            pltpu.SemaphoreType.DMA,
            pltpu.SemaphoreType.DMA,
        ],
    )(hlo, hhi, eal, src, dst)


def _stage3_body(x_ref, hlo_ref, hhi_ref, asrc_ref, adst_ref, sc0_ref, sc1_ref,
                 dnm0_ref, dnm1_ref, exp_ref, bias_ref, lnw_ref, lnb_ref,
                 out_ref):
    al = asrc_ref[:, :16] + adst_ref[:, :16]
    el = jnp.exp(jnp.maximum(al, 0.2 * al))
    exp_mat = exp_ref[...]
    el_exp = jnp.dot(el, exp_mat, preferred_element_type=jnp.float32)
    dn = dnm0_ref[:, :16] + dnm1_ref[:, :16] + el
    dn_exp = jnp.dot(dn, exp_mat, preferred_element_type=jnp.float32)
    h = jnp.concatenate([hlo_ref[...], hhi_ref[...]], axis=-1)
    acc = jnp.concatenate([sc0_ref[...], sc1_ref[...]], axis=-1)
    out = (acc + el_exp * h) / (dn_exp + 1e-16)
    out = out + bias_ref[...] + x_ref[...]
    mu = jnp.mean(out, axis=-1, keepdims=True)
    var = jnp.mean((out - mu) ** 2, axis=-1, keepdims=True)
    out = (out - mu) * jax.lax.rsqrt(var + 1e-5)
    out_ref[...] = out * lnw_ref[...] + lnb_ref[...]


def _stage3(x, hlo, hhi, asrc_p, adst_p, sc0, sc1, dnm0, dnm1, exp_mat,
            bias, ln_w, ln_b, bn):
    n, d = x.shape
    grid = (n // bn,)
    vec = lambda i: (0,)
    return pl.pallas_call(
        _stage3_body,
        grid=grid,
        in_specs=[
            pl.BlockSpec((bn, d), lambda i: (i, 0)),
            pl.BlockSpec((bn, 128), lambda i: (i, 0)),
            pl.BlockSpec((bn, 128), lambda i: (i, 0)),
            pl.BlockSpec((bn, 128), lambda i: (i, 0)),
            pl.BlockSpec((bn, 128), lambda i: (i, 0)),
            pl.BlockSpec((bn, 128), lambda i: (i, 0)),
            pl.BlockSpec((bn, 128), lambda i: (i, 0)),
            pl.BlockSpec((bn, 128), lambda i: (i, 0)),
            pl.BlockSpec((bn, 128), lambda i: (i, 0)),
            pl.BlockSpec((16, d), lambda i: (0, 0)),
            pl.BlockSpec((d,), vec),
            pl.BlockSpec((d,), vec),
            pl.BlockSpec((d,), vec),
        ],
        out_specs=pl.BlockSpec((bn, d), lambda i: (i, 0)),
        out_shape=jax.ShapeDtypeStruct((n, d), jnp.float32),
    )(x, hlo, hhi, asrc_p, adst_p, sc0, sc1, dnm0, dnm1, exp_mat,
      bias, ln_w, ln_b)


def kernel(x, edge_index, W, att_src, att_dst, bias, ln_w, ln_b):
    n, d_in = x.shape
    d_out = W.shape[1]

    eye_h = jnp.eye(H, dtype=jnp.float32)
    As = (att_src[:, :, None] * eye_h[:, None, :]).reshape(d_out, H)
    Ad = (att_dst[:, :, None] * eye_h[:, None, :]).reshape(d_out, H)
    pad = jnp.zeros((d_out, 128 - H), jnp.float32)
    As128 = jnp.concatenate([As, pad], axis=1)
    Ad128 = jnp.concatenate([Ad, pad], axis=1)
    head_of_col = (jnp.arange(d_out, dtype=jnp.int32) // C)[None, :]
    exp_mat = (jnp.arange(16, dtype=jnp.int32)[:, None] == head_of_col
               ).astype(jnp.float32)

    src = edge_index[0]
    dst = edge_index[1]

    bn = 1000
    hlo, hhi, asrc_p, adst_p = _stage1(x, W, As128, Ad128, bn)
    eal, dnm0, dnm1 = _stage2a(asrc_p, adst_p, src, dst)
    sc0, sc1 = _stage2b(hlo, hhi, eal, src, dst)
    return _stage3(x, hlo, hhi, asrc_p, adst_p, sc0, sc1, dnm0, dnm1, exp_mat,
                   bias, ln_w, ln_b, bn)


# speedup vs baseline: 1.3363x; 1.1432x over previous
import functools

import jax
import jax.numpy as jnp
from jax import lax
from jax.experimental import pallas as pl
from jax.experimental.pallas import tpu as pltpu
from jax.experimental.pallas import tpu_sc as plsc

NC = 2
NS = 16
LANES = 16

H = 8
C = 32
KA = 40
KB = 80


def _maybe_when(cond, fn):
    if isinstance(cond, bool):
        if cond:
            fn()
    else:
        pl.when(cond)(fn)


def _ge(a, b):
    return a >= b


def _lt(a, b):
    return a < b


def _bcast_lane(v, lane):
    return lax.gather(
        v, jnp.full((LANES, 1), lane, jnp.int32),
        lax.GatherDimensionNumbers(offset_dims=(),
                                   collapsed_slice_dims=(0,),
                                   start_index_map=(0,)),
        slice_sizes=(1,),
        mode=lax.GatherScatterMode.PROMISE_IN_BOUNDS)


def _zero_rows(ref, nrows, width):
    @pl.loop(0, nrows)
    def _z(i):
        z = jnp.zeros((LANES,), jnp.float32)
        for j in range(width // LANES):
            ref[i, pl.ds(j * LANES, LANES)] = z


def _stage1_body(x_ref, w_ref, as_ref, ad_ref,
                 hlo_ref, hhi_ref, asrc_ref, adst_ref):
    h = jnp.dot(x_ref[...], w_ref[...], preferred_element_type=jnp.float32)
    hlo_ref[...] = h[:, :128]
    hhi_ref[...] = h[:, 128:]
    asrc_ref[...] = jnp.dot(h, as_ref[...], preferred_element_type=jnp.float32)
    adst_ref[...] = jnp.dot(h, ad_ref[...], preferred_element_type=jnp.float32)


def _stage1(x, W, As128, Ad128, bn):
    n, d_in = x.shape
    d_out = W.shape[1]
    grid = (n // bn,)
    return pl.pallas_call(
        _stage1_body,
        grid=grid,
        in_specs=[
            pl.BlockSpec((bn, d_in), lambda i: (i, 0)),
            pl.BlockSpec((d_in, d_out), lambda i: (0, 0)),
            pl.BlockSpec((d_out, 128), lambda i: (0, 0)),
            pl.BlockSpec((d_out, 128), lambda i: (0, 0)),
        ],
        out_specs=[
            pl.BlockSpec((bn, 128), lambda i: (i, 0)),
            pl.BlockSpec((bn, 128), lambda i: (i, 0)),
            pl.BlockSpec((bn, 128), lambda i: (i, 0)),
            pl.BlockSpec((bn, 128), lambda i: (i, 0)),
        ],
        out_shape=[
            jax.ShapeDtypeStruct((n, 128), jnp.float32),
            jax.ShapeDtypeStruct((n, 128), jnp.float32),
            jax.ShapeDtypeStruct((n, 128), jnp.float32),
            jax.ShapeDtypeStruct((n, 128), jnp.float32),
        ],
    )(x, W, As128, Ad128)


def _logit_body(n, e, ch, rows_main,
                asrc, adst, src, dst,
                eal, dnm0, dnm1,
                sidx0, sidx1, didx0, didx1, didx2, didx3, sa0, sa1, da0, da1,
                ebuf0, ebuf1, pbuf0, pbuf1, dnm,
                isem0, isem1, gsem0, gsem1, wsem0, wsem1, ssem0, ssem1):
    c = lax.axis_index("c")
    s = lax.axis_index("s")
    w = c * NS + s
    rows_last = n - rows_main * (NS - 1)
    sidx = (sidx0, sidx1)
    didx = (didx0, didx1, didx2, didx3)
    sa = (sa0, sa1)
    da = (da0, da1)
    ebuf = (ebuf0, ebuf1)
    pbuf = (pbuf0, pbuf1)
    isem = (isem0, isem1)
    gsem = (gsem0, gsem1)
    wsem = (wsem0, wsem1)
    ssem = (ssem0, ssem1)

    _zero_rows(pbuf0, KA, 128)
    _zero_rows(pbuf1, KA, 128)
    row0 = s * rows_main

    def _zero_spmem(nrows):
        off = 0
        while off < nrows:
            step = min(KA, nrows - off)
            pltpu.sync_copy(pbuf0.at[pl.ds(0, step)],
                            dnm.at[pl.ds(row0 + off, step)])
            off += step

    @pl.when(s < NS - 1)
    def _():
        _zero_spmem(rows_main)

    @pl.when(s == NS - 1)
    def _():
        _zero_spmem(rows_last)

    plsc.subcore_barrier()

    ep = e // (NC * NS)

    def _idx_copies(i, p, r):
        base = w * ep + i * KA
        return (pltpu.make_async_copy(src.at[pl.ds(base, KA)], sidx[p],
                                      isem[p]),
                pltpu.make_async_copy(dst.at[pl.ds(base, KA)], didx[r],
                                      isem[p]))

    def _issue_idx(i, p, r):
        for d in _idx_copies(i, p, r):
            d.start()

    def _wait_idx(i, p, r):
        for d in _idx_copies(i, p, r):
            d.wait()

    def _gather_copies(p, r):
        return (pltpu.make_async_copy(asrc.at[sidx[p]], sa[p], gsem[p]),
                pltpu.make_async_copy(adst.at[didx[r]], da[p], gsem[p]))

    def _issue_gather(p, r):
        for d in _gather_copies(p, r):
            d.start()

    def _wait_gather(p, r):
        for d in _gather_copies(p, r):
            d.wait()

    def _eal_write(i, p):
        base = w * ep + i * KA
        return pltpu.make_async_copy(ebuf[p], eal.at[pl.ds(base, KA)],
                                     wsem[p])

    def _wait_scatter(p, r):
        pltpu.make_async_copy(pbuf[p], dnm.at[didx[r]], ssem[p]).wait()

    def _body(i, k):
        p = k % 2
        q = 1 - p
        def _pre():
            _wait_idx(i + 1, q, (k + 1) % 4)
            _maybe_when(_ge(i, 1), lambda: _wait_scatter(q, (k - 1) % 4))
            _issue_gather(q, (k + 1) % 4)

        _maybe_when(_lt(i + 1, ch), _pre)
        _wait_gather(p, k)
        _maybe_when(_ge(i, 2), lambda: _eal_write(i - 2, p).wait())

        @plsc.parallel_loop(0, KA, unroll=2)
        def _edge(j):
            al = sa[p][j, pl.ds(0, LANES)] + da[p][j, pl.ds(0, LANES)]
            ea = jnp.exp(jnp.maximum(al, 0.2 * al))
            ebuf[p][j, :] = ea
            pbuf[p][j, pl.ds(0, LANES)] = ea

        _eal_write(i, p).start()
        pltpu.async_copy(pbuf[p], dnm.at[didx[k]], ssem[p], add=True)
        _maybe_when(_lt(i + 2, ch), lambda: _issue_idx(i + 2, p, (k + 2) % 4))

    _issue_idx(0, 0, 0)
    _issue_idx(1, 1, 1)
    _wait_idx(0, 0, 0)
    _issue_gather(0, 0)

    @pl.loop(0, ch // 4)
    def _grp(g):
        i0 = g * 4
        for k in range(4):
            _body(i0 + k, k)

    for rem in range(ch - ch % 4, ch):
        _body(rem, rem % 4)

    for last in (ch - 2, ch - 1):
        if last >= 0:
            _eal_write(last, last % 2).wait()
            _wait_scatter(last % 2, last % 4)

    plsc.subcore_barrier()

    def _writeout(nrows):
        @pl.when(c == 0)
        def _():
            pltpu.sync_copy(dnm.at[pl.ds(row0, nrows)],
                            dnm0.at[pl.ds(row0, nrows)])

        @pl.when(c == 1)
        def _():
            pltpu.sync_copy(dnm.at[pl.ds(row0, nrows)],
                            dnm1.at[pl.ds(row0, nrows)])

    @pl.when(s < NS - 1)
    def _():
        _writeout(rows_main)

    @pl.when(s == NS - 1)
    def _():
        _writeout(rows_last)


def _stage2a(asrc_p, adst_p, src, dst):
    n = asrc_p.shape[0]
    e = src.shape[0]
    ep = e // (NC * NS)
    ch = ep // KA
    rows_main = ((n // NS) // 8) * 8
    mesh = plsc.VectorSubcoreMesh(core_axis_name="c", subcore_axis_name="s",
                                  num_cores=NC, num_subcores=NS)
    body = functools.partial(_logit_body, n, e, ch, rows_main)
    return pl.kernel(
        body,
        out_type=[
            jax.ShapeDtypeStruct((e, 16), jnp.float32),
            jax.ShapeDtypeStruct((n, 128), jnp.float32),
            jax.ShapeDtypeStruct((n, 128), jnp.float32),
        ],
        mesh=mesh,
        scratch_types=[
            pltpu.VMEM((KA,), jnp.int32),
            pltpu.VMEM((KA,), jnp.int32),
            pltpu.VMEM((KA,), jnp.int32),
            pltpu.VMEM((KA,), jnp.int32),
            pltpu.VMEM((KA,), jnp.int32),
            pltpu.VMEM((KA,), jnp.int32),
            pltpu.VMEM((KA, 128), jnp.float32),
            pltpu.VMEM((KA, 128), jnp.float32),
            pltpu.VMEM((KA, 128), jnp.float32),
            pltpu.VMEM((KA, 128), jnp.float32),
            pltpu.VMEM((KA, 16), jnp.float32),
            pltpu.VMEM((KA, 16), jnp.float32),
            pltpu.VMEM((KA, 128), jnp.float32),
            pltpu.VMEM((KA, 128), jnp.float32),
            pltpu.VMEM_SHARED((n, 128), jnp.float32),
            pltpu.SemaphoreType.DMA,
            pltpu.SemaphoreType.DMA,
            pltpu.SemaphoreType.DMA,
            pltpu.SemaphoreType.DMA,
            pltpu.SemaphoreType.DMA,
            pltpu.SemaphoreType.DMA,
            pltpu.SemaphoreType.DMA,
            pltpu.SemaphoreType.DMA,
        ],
    )(asrc_p, adst_p, src, dst)


def _msg_body(n, e, ch, rows_main,
              hlo, hhi, eal, src, dst,
              out0, out1,
              sidx0, sidx1, didx0, didx1, didx2, didx3,
              hrow0, hrow1, ebuf0, ebuf1, acc,
              isem0, isem1, gsem0, gsem1, ssem0, ssem1):
    c = lax.axis_index("c")
    s = lax.axis_index("s")
    rows_last = n - rows_main * (NS - 1)
    sidx = (sidx0, sidx1)
    didx = (didx0, didx1, didx2, didx3)
    hrow = (hrow0, hrow1)
    ebuf = (ebuf0, ebuf1)
    isem = (isem0, isem1)
    gsem = (gsem0, gsem1)
    ssem = (ssem0, ssem1)

    _zero_rows(hrow0, KB, 128)
    row0 = s * rows_main

    def _zero_spmem(nrows):
        off = 0
        while off < nrows:
            step = min(KB, nrows - off)
            pltpu.sync_copy(hrow0.at[pl.ds(0, step)],
                            acc.at[pl.ds(row0 + off, step)])
            off += step

    @pl.when(s < NS - 1)
    def _():
        _zero_spmem(rows_main)

    @pl.when(s == NS - 1)
    def _():
        _zero_spmem(rows_last)

    plsc.subcore_barrier()

    ep = e // NS

    def _idx_copies(i, p, r):
        base = s * ep + i * KB
        return (pltpu.make_async_copy(src.at[pl.ds(base, KB)], sidx[p],
                                      isem[p]),
                pltpu.make_async_copy(dst.at[pl.ds(base, KB)], didx[r],
                                      isem[p]),
                pltpu.make_async_copy(eal.at[pl.ds(base, KB)], ebuf[p],
                                      isem[p]))

    def _issue_idx(i, p, r):
        for d in _idx_copies(i, p, r):
            d.start()

    def _wait_idx(i, p, r):
        for d in _idx_copies(i, p, r):
            d.wait()

    def _gather_copies(p):
        return (pltpu.make_async_copy(hlo.at[sidx[p]], hrow[p], gsem[p]),
                pltpu.make_async_copy(hhi.at[sidx[p]], hrow[p], gsem[p]))

    def _issue_gather(p):
        lo, hi = _gather_copies(p)

        @pl.when(c == 0)
        def _():
            lo.start()

        @pl.when(c == 1)
        def _():
            hi.start()

    def _wait_gather(p):
        _gather_copies(p)[0].wait()

    def _wait_scatter(p, r):
        pltpu.make_async_copy(hrow[p], acc.at[didx[r]], ssem[p]).wait()

    def _body(i, k):
        p = k % 2
        q = 1 - p
        def _pre():
            _wait_idx(i + 1, q, (k + 1) % 4)
            _maybe_when(_ge(i, 1), lambda: _wait_scatter(q, (k - 1) % 4))
            _issue_gather(q)

        _maybe_when(_lt(i + 1, ch), _pre)
        _wait_gather(p)

        @plsc.parallel_loop(0, KB, unroll=2)
        def _edge(j):
            ea = ebuf[p][j, :]
            for hd in range(4):
                wv = _bcast_lane(ea, 4 * c + hd)
                col = hd * C
                hrow[p][j, pl.ds(col, LANES)] = (
                    hrow[p][j, pl.ds(col, LANES)] * wv)
                hrow[p][j, pl.ds(col + LANES, LANES)] = (
                    hrow[p][j, pl.ds(col + LANES, LANES)] * wv)

        pltpu.async_copy(hrow[p], acc.at[didx[k]], ssem[p], add=True)
        _maybe_when(_lt(i + 2, ch), lambda: _issue_idx(i + 2, p, (k + 2) % 4))

    _issue_idx(0, 0, 0)
    _issue_idx(1, 1, 1)
    _wait_idx(0, 0, 0)
    _issue_gather(0)

    @pl.loop(0, ch // 4)
    def _grp(g):
        i0 = g * 4
        for k in range(4):
            _body(i0 + k, k)

    for rem in range(ch - ch % 4, ch):
        _body(rem, rem % 4)

    for last in (ch - 2, ch - 1):
        if last >= 0:
            _wait_scatter(last % 2, last % 4)

    plsc.subcore_barrier()

    def _writeout(nrows):
        @pl.when(c == 0)
        def _():
            pltpu.sync_copy(acc.at[pl.ds(row0, nrows)],
                            out0.at[pl.ds(row0, nrows)])

        @pl.when(c == 1)
        def _():
            pltpu.sync_copy(acc.at[pl.ds(row0, nrows)],
                            out1.at[pl.ds(row0, nrows)])

    @pl.when(s < NS - 1)
    def _():
        _writeout(rows_main)

    @pl.when(s == NS - 1)
    def _():
        _writeout(rows_last)


def _stage2b(hlo, hhi, eal, src, dst):
    n = hlo.shape[0]
    e = src.shape[0]
    ep = e // NS
    ch = ep // KB
    rows_main = ((n // NS) // 8) * 8
    mesh = plsc.VectorSubcoreMesh(core_axis_name="c", subcore_axis_name="s",
                                  num_cores=NC, num_subcores=NS)
    body = functools.partial(_msg_body, n, e, ch, rows_main)
    return pl.kernel(
        body,
        out_type=[
            jax.ShapeDtypeStruct((n, 128), jnp.float32),
            jax.ShapeDtypeStruct((n, 128), jnp.float32),
        ],
        mesh=mesh,
        scratch_types=[
            pltpu.VMEM((KB,), jnp.int32),
            pltpu.VMEM((KB,), jnp.int32),
            pltpu.VMEM((KB,), jnp.int32),
            pltpu.VMEM((KB,), jnp.int32),
            pltpu.VMEM((KB,), jnp.int32),
            pltpu.VMEM((KB,), jnp.int32),
            pltpu.VMEM((KB, 128), jnp.float32),
            pltpu.VMEM((KB, 128), jnp.float32),
            pltpu.VMEM((KB, 16), jnp.float32),
            pltpu.VMEM((KB, 16), jnp.float32),
            pltpu.VMEM_SHARED((n, 128), jnp.float32),
            pltpu.SemaphoreType.DMA,
            pltpu.SemaphoreType.DMA,
            pltpu.SemaphoreType.DMA,
            pltpu.SemaphoreType.DMA,
            pltpu.SemaphoreType.DMA,
            pltpu.SemaphoreType.DMA,
        ],
    )(hlo, hhi, eal, src, dst)


def _stage3_body(x_ref, hlo_ref, hhi_ref, asrc_ref, adst_ref, sc0_ref, sc1_ref,
                 dnm0_ref, dnm1_ref, exp_ref, bias_ref, lnw_ref, lnb_ref,
                 out_ref):
    al = asrc_ref[:, :16] + adst_ref[:, :16]
    el = jnp.exp(jnp.maximum(al, 0.2 * al))
    exp_mat = exp_ref[...]
    el_exp = jnp.dot(el, exp_mat, preferred_element_type=jnp.float32)
    dn = dnm0_ref[:, :16] + dnm1_ref[:, :16] + el
    dn_exp = jnp.dot(dn, exp_mat, preferred_element_type=jnp.float32)
    h = jnp.concatenate([hlo_ref[...], hhi_ref[...]], axis=-1)
    acc = jnp.concatenate([sc0_ref[...], sc1_ref[...]], axis=-1)
    out = (acc + el_exp * h) / (dn_exp + 1e-16)
    out = out + bias_ref[...] + x_ref[...]
    mu = jnp.mean(out, axis=-1, keepdims=True)
    var = jnp.mean((out - mu) ** 2, axis=-1, keepdims=True)
    out = (out - mu) * jax.lax.rsqrt(var + 1e-5)
    out_ref[...] = out * lnw_ref[...] + lnb_ref[...]


def _stage3(x, hlo, hhi, asrc_p, adst_p, sc0, sc1, dnm0, dnm1, exp_mat,
            bias, ln_w, ln_b, bn):
    n, d = x.shape
    grid = (n // bn,)
    vec = lambda i: (0,)
    return pl.pallas_call(
        _stage3_body,
        grid=grid,
        in_specs=[
            pl.BlockSpec((bn, d), lambda i: (i, 0)),
            pl.BlockSpec((bn, 128), lambda i: (i, 0)),
            pl.BlockSpec((bn, 128), lambda i: (i, 0)),
            pl.BlockSpec((bn, 128), lambda i: (i, 0)),
            pl.BlockSpec((bn, 128), lambda i: (i, 0)),
            pl.BlockSpec((bn, 128), lambda i: (i, 0)),
            pl.BlockSpec((bn, 128), lambda i: (i, 0)),
            pl.BlockSpec((bn, 128), lambda i: (i, 0)),
            pl.BlockSpec((bn, 128), lambda i: (i, 0)),
            pl.BlockSpec((16, d), lambda i: (0, 0)),
            pl.BlockSpec((d,), vec),
            pl.BlockSpec((d,), vec),
            pl.BlockSpec((d,), vec),
        ],
        out_specs=pl.BlockSpec((bn, d), lambda i: (i, 0)),
        out_shape=jax.ShapeDtypeStruct((n, d), jnp.float32),
    )(x, hlo, hhi, asrc_p, adst_p, sc0, sc1, dnm0, dnm1, exp_mat,
      bias, ln_w, ln_b)


def kernel(x, edge_index, W, att_src, att_dst, bias, ln_w, ln_b):
    n, d_in = x.shape
    d_out = W.shape[1]

    eye_h = jnp.eye(H, dtype=jnp.float32)
    As = (att_src[:, :, None] * eye_h[:, None, :]).reshape(d_out, H)
    Ad = (att_dst[:, :, None] * eye_h[:, None, :]).reshape(d_out, H)
    pad = jnp.zeros((d_out, 128 - H), jnp.float32)
    As128 = jnp.concatenate([As, pad], axis=1)
    Ad128 = jnp.concatenate([Ad, pad], axis=1)
    head_of_col = (jnp.arange(d_out, dtype=jnp.int32) // C)[None, :]
    exp_mat = (jnp.arange(16, dtype=jnp.int32)[:, None] == head_of_col
               ).astype(jnp.float32)

    src = edge_index[0]
    dst = edge_index[1]

    bn = 1000
    hlo, hhi, asrc_p, adst_p = _stage1(x, W, As128, Ad128, bn)
    eal, dnm0, dnm1 = _stage2a(asrc_p, adst_p, src, dst)
    sc0, sc1 = _stage2b(hlo, hhi, eal, src, dst)
    return _stage3(x, hlo, hhi, asrc_p, adst_p, sc0, sc1, dnm0, dnm1, exp_mat,
                   bias, ln_w, ln_b, bn)
